# baseline (speedup 1.0000x reference)
import functools

import jax
import jax.numpy as jnp
from jax import lax
from jax.experimental import pallas as pl
from jax.experimental.pallas import tpu as pltpu
from jax.experimental.pallas import tpu_sc as plsc

_NC, _NS, _L = 2, 16, 16
_NW = _NC * _NS
_H = 64
_EPS = 1e-12
_CHUNK = 128


def _lane_gather(v, idx):
    dnums = lax.GatherDimensionNumbers(
        offset_dims=(), collapsed_slice_dims=(0,), start_index_map=(0,))
    return lax.gather(v, idx[:, None], dnums, (1,),
                      mode=lax.GatherScatterMode.PROMISE_IN_BOUNDS)


def _lane_sum(v):
    lanes = lax.iota(jnp.int32, _L)
    for m in (1, 2, 4, 8):
        v = v + _lane_gather(v, lanes ^ m)
    return v


def _rsqrt(x):
    xi = lax.bitcast_convert_type(x, jnp.int32)
    yi = jnp.int32(0x5F3759DF) - (xi >> 1)
    y = lax.bitcast_convert_type(yi, jnp.float32)
    for _ in range(3):
        y = y * (jnp.float32(1.5) - jnp.float32(0.5) * x * y * y)
    return y


@functools.lru_cache(maxsize=None)
def _build(n_tokens, chunk, interpret=False):
    tok_per_w = n_tokens // _NW
    n_chunks = tok_per_w // chunk
    mesh = plsc.VectorSubcoreMesh(core_axis_name="c", subcore_axis_name="s",
                                  num_cores=_NC, num_subcores=_NS)

    def body(ids_hbm, tt_hbm, pos_hbm, lt_hbm,
             concept_hbm, age_hbm, seg_hbm, type_hbm, gamma_hbm, beta_hbm,
             out_hbm,
             ids_v, tt_v, pos_v, lt_v, crow, srow, arow, trow, gb_v, sem):
        wid = lax.axis_index("s") * _NC + lax.axis_index("c")
        base = wid * tok_per_w

        pltpu.sync_copy(gamma_hbm, gb_v.at[0])
        pltpu.sync_copy(beta_hbm, gb_v.at[1])
        gammas = [gb_v[0, pl.ds(k * _L, _L)] for k in range(_H // _L)]
        betas = [gb_v[1, pl.ds(k * _L, _L)] for k in range(_H // _L)]

        def chunk_body(ci, carry):
            off = base + ci * chunk
            pltpu.sync_copy(ids_hbm.at[pl.ds(off, chunk)], ids_v)
            pltpu.sync_copy(tt_hbm.at[pl.ds(off, chunk)], tt_v)
            pltpu.sync_copy(pos_hbm.at[pl.ds(off, chunk)], pos_v)
            pltpu.sync_copy(lt_hbm.at[pl.ds(off, chunk)], lt_v)
            pltpu.async_copy(concept_hbm.at[ids_v], crow, sem).wait()
            pltpu.async_copy(seg_hbm.at[tt_v], srow, sem).wait()
            pltpu.async_copy(age_hbm.at[pos_v], arow, sem).wait()
            pltpu.async_copy(type_hbm.at[lt_v], trow, sem).wait()

            def tok(t, tcarry):
                vs = []
                for k in range(_H // _L):
                    sl = pl.ds(k * _L, _L)
                    vs.append(crow[t, sl] + srow[t, sl]
                              + arow[t, sl] + trow[t, sl])
                tot = (vs[0] + vs[1]) + (vs[2] + vs[3])
                sq = (vs[0] * vs[0] + vs[1] * vs[1]) \
                    + (vs[2] * vs[2] + vs[3] * vs[3])
                s1 = _lane_sum(tot)
                s2 = _lane_sum(sq)
                mean = s1 * jnp.float32(1.0 / _H)
                var = s2 * jnp.float32(1.0 / _H) - mean * mean
                r = _rsqrt(var + jnp.float32(_EPS))
                for k in range(_H // _L):
                    crow[t, pl.ds(k * _L, _L)] = \
                        (vs[k] - mean) * r * gammas[k] + betas[k]
                return tcarry

            lax.fori_loop(0, chunk, tok, 0)
            pltpu.sync_copy(crow, out_hbm.at[pl.ds(off, chunk)])
            return carry

        lax.fori_loop(0, n_chunks, chunk_body, 0)

    return pl.kernel(
        body,
        out_type=jax.ShapeDtypeStruct((n_tokens, _H), jnp.float32),
        mesh=mesh,
        scratch_types=(
            [pltpu.VMEM((chunk,), jnp.int32)] * 4
            + [pltpu.VMEM((chunk, _H), jnp.float32)] * 4
            + [pltpu.VMEM((2, _H), jnp.float32), pltpu.SemaphoreType.DMA]
        ),
        compiler_params=pltpu.CompilerParams(use_tc_tiling_on_sc=False),
        interpret=interpret,
    )


def kernel(input_ids, token_type_ids, position_ids, label_type_ids,
           concept_table, age_table, segment_table, type_table,
           ln_gamma, ln_beta):
    b, s = input_ids.shape
    n = b * s
    f = _build(n, _CHUNK)
    out = f(input_ids.reshape(n).astype(jnp.int32),
            token_type_ids.reshape(n).astype(jnp.int32),
            position_ids.reshape(n).astype(jnp.int32),
            label_type_ids.reshape(n).astype(jnp.int32),
            concept_table, age_table, segment_table, type_table,
            ln_gamma, ln_beta)
    return out.reshape(b, s, _H)


# speedup vs baseline: 7.4951x; 7.4951x over previous
import functools

import jax
import jax.numpy as jnp
from jax import lax
from jax.experimental import pallas as pl
from jax.experimental.pallas import tpu as pltpu
from jax.experimental.pallas import tpu_sc as plsc

_NC, _NS, _L = 2, 16, 16
_NW = _NC * _NS
_H = 64
_EPS = 1e-12
_CHUNK = 128
_NSEG, _NAGE, _NTYP = 2, 180, 20
_NCOMB = _NSEG * _NAGE * _NTYP


def _lane_gather(v, idx):
    dnums = lax.GatherDimensionNumbers(
        offset_dims=(), collapsed_slice_dims=(0,), start_index_map=(0,))
    return lax.gather(v, idx[:, None], dnums, (1,),
                      mode=lax.GatherScatterMode.PROMISE_IN_BOUNDS)


def _lane_sum(v):
    lanes = lax.iota(jnp.int32, _L)
    for m in (1, 2, 4, 8):
        v = v + _lane_gather(v, lanes ^ m)
    return v


def _rsqrt(x):
    xi = lax.bitcast_convert_type(x, jnp.int32)
    yi = jnp.int32(0x5F3759DF) - (xi >> 1)
    y = lax.bitcast_convert_type(yi, jnp.float32)
    for _ in range(3):
        y = y * (jnp.float32(1.5) - jnp.float32(0.5) * x * y * y)
    return y


def _combined_table(age, seg, typ):
    def body(seg_ref, age_ref, typ_ref, out_ref):
        s = seg_ref[...]
        a = age_ref[...]
        t = typ_ref[...]
        ap = (s[:, None, :] + a[None, :, :]).reshape(_NSEG * _NAGE, _H)
        out_ref[...] = (ap[:, None, :] + t[None, :, :]).reshape(_NCOMB, _H)
    return pl.pallas_call(
        body,
        out_shape=jax.ShapeDtypeStruct((_NCOMB, _H), jnp.float32),
    )(seg, age, typ)


@functools.lru_cache(maxsize=None)
def _build(n_tokens, chunk, interpret=False):
    tok_per_w = n_tokens // _NW
    n_chunks = tok_per_w // chunk
    assert n_chunks % 2 == 0
    mesh = plsc.VectorSubcoreMesh(core_axis_name="c", subcore_axis_name="s",
                                  num_cores=_NC, num_subcores=_NS)

    def body(ids_hbm, tt_hbm, pos_hbm, lt_hbm,
             concept_hbm, comb_hbm, gamma_hbm, beta_hbm,
             out_hbm,
             ids2, tt_v, pos_v, lt_v, cidx2, crow2, mrow2, gb_v,
             sem_g, sem_o):
        wid = lax.axis_index("s") * _NC + lax.axis_index("c")
        base = wid * tok_per_w

        pltpu.sync_copy(gamma_hbm, gb_v.at[0])
        pltpu.sync_copy(beta_hbm, gb_v.at[1])
        gammas = [gb_v[0, pl.ds(k * _L, _L)] for k in range(_H // _L)]
        betas = [gb_v[1, pl.ds(k * _L, _L)] for k in range(_H // _L)]

        def load_idx(c, slot):
            off = base + c * chunk
            pltpu.sync_copy(ids_hbm.at[pl.ds(off, chunk)], ids2.at[slot])
            pltpu.sync_copy(tt_hbm.at[pl.ds(off, chunk)], tt_v)
            pltpu.sync_copy(pos_hbm.at[pl.ds(off, chunk)], pos_v)
            pltpu.sync_copy(lt_hbm.at[pl.ds(off, chunk)], lt_v)
            for j in range(chunk // _L):
                sl = pl.ds(j * _L, _L)
                cidx2[slot, sl] = (tt_v[sl] * jnp.int32(_NAGE * _NTYP)
                                   + pos_v[sl] * jnp.int32(_NTYP) + lt_v[sl])

        def start_gathers(slot):
            pltpu.make_async_copy(concept_hbm.at[ids2.at[slot]],
                                  crow2.at[slot], sem_g.at[slot]).start()
            pltpu.make_async_copy(comb_hbm.at[cidx2.at[slot]],
                                  mrow2.at[slot], sem_g.at[slot]).start()

        def wait_gathers(slot):
            pltpu.make_async_copy(concept_hbm.at[ids2.at[slot]],
                                  crow2.at[slot], sem_g.at[slot]).wait()
            pltpu.make_async_copy(comb_hbm.at[cidx2.at[slot]],
                                  mrow2.at[slot], sem_g.at[slot]).wait()

        def out_copy_desc(c, slot):
            off = base + c * chunk
            return pltpu.make_async_copy(
                crow2.at[slot], out_hbm.at[pl.ds(off, chunk)], sem_o.at[slot])

        def compute(slot):
            def tok(t, tcarry):
                vs = []
                for k in range(_H // _L):
                    sl = pl.ds(k * _L, _L)
                    vs.append(crow2[slot, t, sl] + mrow2[slot, t, sl])
                tot = (vs[0] + vs[1]) + (vs[2] + vs[3])
                sq = (vs[0] * vs[0] + vs[1] * vs[1]) \
                    + (vs[2] * vs[2] + vs[3] * vs[3])
                s1 = _lane_sum(tot)
                s2 = _lane_sum(sq)
                mean = s1 * jnp.float32(1.0 / _H)
                var = s2 * jnp.float32(1.0 / _H) - mean * mean
                r = _rsqrt(var + jnp.float32(_EPS))
                for k in range(_H // _L):
                    crow2[slot, t, pl.ds(k * _L, _L)] = \
                        (vs[k] - mean) * r * gammas[k] + betas[k]
                return tcarry
            lax.fori_loop(0, chunk, tok, 0)

        load_idx(0, 0)
        start_gathers(0)

        def outer(i, carry):
            for b in (0, 1):
                c = i * 2 + b
                nb = 1 - b

                @pl.when(c + 1 < n_chunks)
                def _():
                    load_idx(c + 1, nb)

                    @pl.when(c >= 1)
                    def _():
                        out_copy_desc(c - 1, nb).wait()
                    start_gathers(nb)

                wait_gathers(b)
                compute(b)
                out_copy_desc(c, b).start()
            return carry

        lax.fori_loop(0, n_chunks // 2, outer, 0)
        out_copy_desc(n_chunks - 2, 0).wait()
        out_copy_desc(n_chunks - 1, 1).wait()

    return pl.kernel(
        body,
        out_type=jax.ShapeDtypeStruct((n_tokens, _H), jnp.float32),
        mesh=mesh,
        scratch_types=(
            [pltpu.VMEM((2, chunk), jnp.int32)]
            + [pltpu.VMEM((chunk,), jnp.int32)] * 3
            + [pltpu.VMEM((2, chunk), jnp.int32)]
            + [pltpu.VMEM((2, chunk, _H), jnp.float32)] * 2
            + [pltpu.VMEM((2, _H), jnp.float32)]
            + [pltpu.SemaphoreType.DMA((2,)), pltpu.SemaphoreType.DMA((2,))]
        ),
        compiler_params=pltpu.CompilerParams(use_tc_tiling_on_sc=False),
        interpret=interpret,
    )


def kernel(input_ids, token_type_ids, position_ids, label_type_ids,
           concept_table, age_table, segment_table, type_table,
           ln_gamma, ln_beta):
    b, s = input_ids.shape
    n = b * s
    comb = _combined_table(age_table, segment_table, type_table)
    f = _build(n, _CHUNK)
    out = f(input_ids.reshape(n).astype(jnp.int32),
            token_type_ids.reshape(n).astype(jnp.int32),
            position_ids.reshape(n).astype(jnp.int32),
            label_type_ids.reshape(n).astype(jnp.int32),
            concept_table, comb, ln_gamma, ln_beta)
    return out.reshape(b, s, _H)


# speedup vs baseline: 10.8635x; 1.4494x over previous
import functools

import jax
import jax.numpy as jnp
from jax import lax
from jax.experimental import pallas as pl
from jax.experimental.pallas import tpu as pltpu
from jax.experimental.pallas import tpu_sc as plsc

_NC, _NS, _L = 2, 16, 16
_NW = _NC * _NS
_H = 64
_EPS = 1e-12
_CHUNK = 128
_NSEG, _NAGE, _NTYP = 2, 180, 20
_NCOMB = _NSEG * _NAGE * _NTYP


def _lane_gather(v, idx):
    dnums = lax.GatherDimensionNumbers(
        offset_dims=(), collapsed_slice_dims=(0,), start_index_map=(0,))
    return lax.gather(v, idx[:, None], dnums, (1,),
                      mode=lax.GatherScatterMode.PROMISE_IN_BOUNDS)


def _lane_sum(v):
    lanes = lax.iota(jnp.int32, _L)
    for m in (1, 2, 4, 8):
        v = v + _lane_gather(v, lanes ^ m)
    return v


def _rsqrt(x):
    xi = lax.bitcast_convert_type(x, jnp.int32)
    yi = jnp.int32(0x5F3759DF) - (xi >> 1)
    y = lax.bitcast_convert_type(yi, jnp.float32)
    for _ in range(3):
        y = y * (jnp.float32(1.5) - jnp.float32(0.5) * x * y * y)
    return y


def _combined_table(age, seg, typ):
    def body(seg_ref, age_ref, typ_ref, out_ref):
        s = seg_ref[...]
        a = age_ref[...]
        t = typ_ref[...]
        ap = (s[:, None, :] + a[None, :, :]).reshape(_NSEG * _NAGE, _H)
        out_ref[...] = (ap[:, None, :] + t[None, :, :]).reshape(_NCOMB, _H)
    return pl.pallas_call(
        body,
        out_shape=jax.ShapeDtypeStruct((_NCOMB, _H), jnp.float32),
    )(seg, age, typ)


@functools.lru_cache(maxsize=None)
def _build(n_tokens, chunk, interpret=False):
    tok_per_w = n_tokens // _NW
    n_chunks = tok_per_w // chunk
    assert n_chunks % 2 == 0
    mesh = plsc.VectorSubcoreMesh(core_axis_name="c", subcore_axis_name="s",
                                  num_cores=_NC, num_subcores=_NS)

    def body(ids_hbm, tt_hbm, pos_hbm, lt_hbm,
             concept_hbm, comb_hbm, gamma_hbm, beta_hbm,
             out_hbm,
             ids2, tt_v, pos_v, lt_v, cidx2, crow2, mrow2, outb2, gb_v,
             sem_g, sem_o):
        wid = lax.axis_index("s") * _NC + lax.axis_index("c")
        base = wid * tok_per_w

        pltpu.sync_copy(gamma_hbm, gb_v.at[0])
        pltpu.sync_copy(beta_hbm, gb_v.at[1])
        gammas = [gb_v[0, pl.ds(k * _L, _L)] for k in range(_H // _L)]
        betas = [gb_v[1, pl.ds(k * _L, _L)] for k in range(_H // _L)]

        def load_idx(c, slot):
            off = base + c * chunk
            pltpu.sync_copy(ids_hbm.at[pl.ds(off, chunk)], ids2.at[slot])
            pltpu.sync_copy(tt_hbm.at[pl.ds(off, chunk)], tt_v)
            pltpu.sync_copy(pos_hbm.at[pl.ds(off, chunk)], pos_v)
            pltpu.sync_copy(lt_hbm.at[pl.ds(off, chunk)], lt_v)
            for j in range(chunk // _L):
                sl = pl.ds(j * _L, _L)
                cidx2[slot, sl] = (tt_v[sl] * jnp.int32(_NAGE * _NTYP)
                                   + pos_v[sl] * jnp.int32(_NTYP) + lt_v[sl])

        def start_gathers(slot):
            pltpu.make_async_copy(concept_hbm.at[ids2.at[slot]],
                                  crow2.at[slot], sem_g.at[slot]).start()
            pltpu.make_async_copy(comb_hbm.at[cidx2.at[slot]],
                                  mrow2.at[slot], sem_g.at[slot]).start()

        def wait_gathers(slot):
            pltpu.make_async_copy(concept_hbm.at[ids2.at[slot]],
                                  crow2.at[slot], sem_g.at[slot]).wait()
            pltpu.make_async_copy(comb_hbm.at[cidx2.at[slot]],
                                  mrow2.at[slot], sem_g.at[slot]).wait()

        def out_copy_desc(c, slot):
            off = base + c * chunk
            return pltpu.make_async_copy(
                outb2.at[slot], out_hbm.at[pl.ds(off, chunk)], sem_o.at[slot])

        def compute(slot):
            def tok(t):
                vs = []
                for k in range(_H // _L):
                    sl = pl.ds(k * _L, _L)
                    vs.append(crow2[slot, t, sl] + mrow2[slot, t, sl])
                tot = (vs[0] + vs[1]) + (vs[2] + vs[3])
                sq = (vs[0] * vs[0] + vs[1] * vs[1]) \
                    + (vs[2] * vs[2] + vs[3] * vs[3])
                s1 = _lane_sum(tot)
                s2 = _lane_sum(sq)
                mean = s1 * jnp.float32(1.0 / _H)
                var = s2 * jnp.float32(1.0 / _H) - mean * mean
                r = _rsqrt(var + jnp.float32(_EPS))
                for k in range(_H // _L):
                    outb2[slot, t, pl.ds(k * _L, _L)] = \
                        (vs[k] - mean) * r * gammas[k] + betas[k]
            plsc.parallel_loop(0, chunk, 1, unroll=4)(tok)

        load_idx(0, 0)
        start_gathers(0)

        def outer(i, carry):
            for b in (0, 1):
                c = i * 2 + b
                nb = 1 - b

                @pl.when(c + 1 < n_chunks)
                def _():
                    load_idx(c + 1, nb)
                    start_gathers(nb)

                wait_gathers(b)

                @pl.when(c >= 2)
                def _():
                    out_copy_desc(c - 2, b).wait()

                compute(b)
                out_copy_desc(c, b).start()
            return carry

        lax.fori_loop(0, n_chunks // 2, outer, 0)
        out_copy_desc(n_chunks - 2, 0).wait()
        out_copy_desc(n_chunks - 1, 1).wait()

    return pl.kernel(
        body,
        out_type=jax.ShapeDtypeStruct((n_tokens, _H), jnp.float32),
        mesh=mesh,
        scratch_types=(
            [pltpu.VMEM((2, chunk), jnp.int32)]
            + [pltpu.VMEM((chunk,), jnp.int32)] * 3
            + [pltpu.VMEM((2, chunk), jnp.int32)]
            + [pltpu.VMEM((2, chunk, _H), jnp.float32)] * 3
            + [pltpu.VMEM((2, _H), jnp.float32)]
            + [pltpu.SemaphoreType.DMA((2,)), pltpu.SemaphoreType.DMA((2,))]
        ),
        compiler_params=pltpu.CompilerParams(use_tc_tiling_on_sc=False),
        interpret=interpret,
    )


def kernel(input_ids, token_type_ids, position_ids, label_type_ids,
           concept_table, age_table, segment_table, type_table,
           ln_gamma, ln_beta):
    b, s = input_ids.shape
    n = b * s
    comb = _combined_table(age_table, segment_table, type_table)
    f = _build(n, _CHUNK)
    out = f(input_ids.reshape(n).astype(jnp.int32),
            token_type_ids.reshape(n).astype(jnp.int32),
            position_ids.reshape(n).astype(jnp.int32),
            label_type_ids.reshape(n).astype(jnp.int32),
            concept_table, comb, ln_gamma, ln_beta)
    return out.reshape(b, s, _H)
